# baseline (speedup 1.0000x reference)
import jax
import jax.numpy as jnp
from jax import lax
from jax.experimental import pallas as pl
from jax.experimental.pallas import tpu as pltpu
from jax.experimental.pallas import tpu_sc as plsc


LAT = 64
PK = 4
LATP = LAT * PK
NC = 2
NS = 16
NW = NC * NS


def _leaky(x):
    return jnp.where(x > 0, x, 0.01 * x)


def _blk(W):
    return jnp.kron(jnp.eye(PK, dtype=W.dtype), W)


def _tile(b):
    return jnp.tile(b, PK).reshape(1, b.shape[0] * PK)


def _bgrp():
    return jnp.kron(jnp.eye(PK, dtype=jnp.float32),
                    jnp.full((LAT, LAT), 1.0 / LAT, jnp.float32))


def _ln_packed(h, bgrp, g, beta):
    mu = jnp.dot(h, bgrp, preferred_element_type=jnp.float32)
    d = h - mu
    v = jnp.dot(d * d, bgrp, preferred_element_type=jnp.float32)
    return d * jax.lax.rsqrt(v + 1e-5) * g + beta


_CHUNK = 128
_GSUB = 4
_GRP = _CHUNK * _GSUB


def _sc_gather_body(tabA, tabB, senders, receivers, ga, gb,
                    sidx, ridx, bufA, bufB, semA, semB, semi):
    c = lax.axis_index("c")
    s = lax.axis_index("s")
    wid = c * NS + s
    E = senders.shape[0]
    ngrp = E // _GRP
    niter = (ngrp + NW - 1) // NW

    def group(j, _):
        g = wid + j * NW

        @pl.when(g < ngrp)
        def _():
            base = g * _GRP
            cpi1 = pltpu.async_copy(senders.at[pl.ds(base, _GRP)], sidx, semi)
            cpi2 = pltpu.async_copy(receivers.at[pl.ds(base, _GRP)], ridx, semi)
            cpi1.wait()
            cpi2.wait()
            cps = []
            for k in range(_GSUB):
                cps.append(pltpu.async_copy(
                    tabA.at[sidx.at[pl.ds(k * _CHUNK, _CHUNK)]],
                    bufA.at[k], semA))
                cps.append(pltpu.async_copy(
                    tabB.at[ridx.at[pl.ds(k * _CHUNK, _CHUNK)]],
                    bufB.at[k], semB))
            for cp in cps:
                cp.wait()
            cpo1 = pltpu.async_copy(bufA, ga.at[pl.ds(g * _GSUB, _GSUB)], semA)
            cpo2 = pltpu.async_copy(bufB, gb.at[pl.ds(g * _GSUB, _GSUB)], semB)
            cpo1.wait()
            cpo2.wait()
        return 0

    lax.fori_loop(0, niter, group, 0)


def _sc_gather(tabA, tabB, senders, receivers):
    E = senders.shape[0]
    mesh = plsc.VectorSubcoreMesh(core_axis_name="c", subcore_axis_name="s")
    out = jax.ShapeDtypeStruct((E // _CHUNK, _CHUNK, LAT), jnp.float32)
    return pl.kernel(
        _sc_gather_body,
        out_type=(out, out),
        mesh=mesh,
        scratch_types=[
            pltpu.VMEM((_GRP,), jnp.int32),
            pltpu.VMEM((_GRP,), jnp.int32),
            pltpu.VMEM((_GSUB, _CHUNK, LAT), jnp.float32),
            pltpu.VMEM((_GSUB, _CHUNK, LAT), jnp.float32),
            pltpu.SemaphoreType.DMA,
            pltpu.SemaphoreType.DMA,
            pltpu.SemaphoreType.DMA,
        ],
        compiler_params=pltpu.CompilerParams(use_tc_tiling_on_sc=False),
    )(tabA, tabB, senders, receivers)


def _sc_scatter_body(vals, receivers, zeros, out, ridx, vbuf, acc, sem):
    c = lax.axis_index("c")
    s = lax.axis_index("s")
    wid = c * NS + s
    nchunk = vals.shape[0]
    E = nchunk * _CHUNK
    N = zeros.shape[0]
    rows = N // NS
    niter = (nchunk + NW - 1) // NW

    pltpu.sync_copy(zeros.at[pl.ds(s * rows, rows)],
                    acc.at[pl.ds(s * rows, rows)])
    plsc.subcore_barrier()

    def chunk(j, _):
        ch = wid + j * NW

        @pl.when(ch < nchunk)
        def _():
            base = ch * _CHUNK
            cpi = pltpu.async_copy(receivers.at[pl.ds(base, _CHUNK)], ridx, sem)
            cpv = pltpu.async_copy(vals.at[ch], vbuf, sem)
            cpi.wait()
            cpv.wait()
            pltpu.sync_copy(vbuf, acc.at[ridx], add=True)
        return 0

    lax.fori_loop(0, niter, chunk, 0)
    plsc.subcore_barrier()
    pltpu.sync_copy(acc.at[pl.ds(s * rows, rows)], out.at[c, s])


def _sc_scatter(vals, receivers, zeros):
    N = zeros.shape[0]
    mesh = plsc.VectorSubcoreMesh(core_axis_name="c", subcore_axis_name="s")
    return pl.kernel(
        _sc_scatter_body,
        out_type=jax.ShapeDtypeStruct((NC, NS, N // NS, LAT), jnp.float32),
        mesh=mesh,
        scratch_types=[
            pltpu.VMEM((_CHUNK,), jnp.int32),
            pltpu.VMEM((_CHUNK, LAT), jnp.float32),
            pltpu.VMEM_SHARED((N, LAT), jnp.float32),
            pltpu.SemaphoreType.DMA,
        ],
        compiler_params=pltpu.CompilerParams(use_tc_tiling_on_sc=False),
    )(vals, receivers, zeros)


def _edge_mlp_body(x1_ref, ga_ref, gb_ref, c_ref, m_ref, w1_ref, b1_ref,
                   w2_ref, b2_ref, w3_ref, b3_ref, bgrp_ref, g_ref, beta_ref,
                   o_ref):
    h = jnp.dot(x1_ref[...], m_ref[...], preferred_element_type=jnp.float32)
    h = h + ga_ref[...] + gb_ref[...] + c_ref[...]
    h = _leaky(h)
    h = _leaky(jnp.dot(h, w1_ref[...], preferred_element_type=jnp.float32) + b1_ref[...])
    h = _leaky(jnp.dot(h, w2_ref[...], preferred_element_type=jnp.float32) + b2_ref[...])
    h = jnp.dot(h, w3_ref[...], preferred_element_type=jnp.float32) + b3_ref[...]
    o_ref[...] = _ln_packed(h, bgrp_ref[...], g_ref[...], beta_ref[...])


def _edge_mlp(x1, ga, gb, c, M, tail_params, ln, block_rows=1000):
    EP, KP = x1.shape
    grid = (EP // block_rows,)
    (w1, b1), (w2, b2), (w3, b3) = tail_params
    g, beta = ln
    row_spec = pl.BlockSpec((block_rows, LATP), lambda i: (i, 0))
    x1_spec = pl.BlockSpec((block_rows, KP), lambda i: (i, 0))
    c_spec = (row_spec if c.shape[0] == EP
              else pl.BlockSpec((1, LATP), lambda i: (0, 0)))
    full = lambda a: pl.BlockSpec(a.shape, lambda i: (0,) * a.ndim)
    small = [_blk(M), _blk(w1), _tile(b1), _blk(w2), _tile(b2),
             _blk(w3), _tile(b3), _bgrp(), _tile(g), _tile(beta)]
    return pl.pallas_call(
        _edge_mlp_body,
        grid=grid,
        in_specs=[x1_spec, row_spec, row_spec, c_spec] + [full(a) for a in small],
        out_specs=row_spec,
        out_shape=jax.ShapeDtypeStruct((EP, LATP), jnp.float32),
    )(x1, ga, gb, c, *small)


def _ce_body(ee_ref, g2a_ref, g2b_ref, wee_ref, bp0_ref, o_ref):
    o_ref[...] = (jnp.dot(ee_ref[...], wee_ref[...],
                          preferred_element_type=jnp.float32)
                  + g2a_ref[...] + g2b_ref[...] + bp0_ref[...])


def _ce_pass(ee, g2a, g2b, W_ee, bp0, block_rows=1000):
    EP = ee.shape[0]
    row_spec = pl.BlockSpec((block_rows, LATP), lambda i: (i, 0))
    return pl.pallas_call(
        _ce_body,
        grid=(EP // block_rows,),
        in_specs=[row_spec, row_spec, row_spec,
                  pl.BlockSpec((LATP, LATP), lambda i: (0, 0)),
                  pl.BlockSpec((1, LATP), lambda i: (0, 0))],
        out_specs=row_spec,
        out_shape=jax.ShapeDtypeStruct((EP, LATP), jnp.float32),
    )(ee, g2a, g2b, _blk(W_ee), _tile(bp0))


def _enc_node_body(nodes_ref, p0_ref, p1_ref, vn_ref, va_ref, b0_ref, w1_ref,
                   b1_ref, w2_ref, b2_ref, w3_ref, b3_ref, bgrp_ref, g_ref,
                   beta_ref, wg0s_ref, wg0r_ref, wsn_ref, wrn_ref, uen_ref,
                   uga_ref, bu0_ref,
                   en_ref, a0_ref, b0out_ref, p2_ref, q2_ref, cn_ref):
    dot = lambda a, b: jnp.dot(a, b, preferred_element_type=jnp.float32)
    agg0 = p0_ref[...] + p1_ref[...]
    h = dot(nodes_ref[...], vn_ref[...]) + dot(agg0, va_ref[...]) + b0_ref[...]
    h = _leaky(h)
    h = _leaky(dot(h, w1_ref[...]) + b1_ref[...])
    h = _leaky(dot(h, w2_ref[...]) + b2_ref[...])
    h = dot(h, w3_ref[...]) + b3_ref[...]
    en = _ln_packed(h, bgrp_ref[...], g_ref[...], beta_ref[...])
    en_ref[...] = en
    p2_ref[...] = dot(en, wg0s_ref[...])
    q2_ref[...] = dot(en, wg0r_ref[...])
    a0_ref[...] = dot(en, wsn_ref[...])
    b0out_ref[...] = dot(en, wrn_ref[...])
    cn_ref[...] = dot(en, uen_ref[...]) + dot(agg0, uga_ref[...]) + bu0_ref[...]


def _step_node_body(ln_ref, p0_ref, p1_ref, cn_ref, uln_ref, uagg_ref,
                    w1_ref, b1_ref, w2_ref, b2_ref, w3_ref, b3_ref,
                    bgrp_ref, g_ref, beta_ref, wsn_ref, wrn_ref,
                    lnout_ref, aout_ref, bout_ref):
    dot = lambda a, b: jnp.dot(a, b, preferred_element_type=jnp.float32)
    agg = p0_ref[...] + p1_ref[...]
    h = dot(ln_ref[...], uln_ref[...]) + dot(agg, uagg_ref[...]) + cn_ref[...]
    h = _leaky(h)
    h = _leaky(dot(h, w1_ref[...]) + b1_ref[...])
    h = _leaky(dot(h, w2_ref[...]) + b2_ref[...])
    h = dot(h, w3_ref[...]) + b3_ref[...]
    ln2 = _ln_packed(h, bgrp_ref[...], g_ref[...], beta_ref[...])
    lnout_ref[...] = ln2
    aout_ref[...] = dot(ln2, wsn_ref[...])
    bout_ref[...] = dot(ln2, wrn_ref[...])


def _dec_body(ln_ref, w0_ref, b0_ref, w1_ref, b1_ref, w2_ref, b2_ref,
              w3_ref, b3_ref, o_ref):
    dot = lambda a, b: jnp.dot(a, b, preferred_element_type=jnp.float32)
    h = _leaky(dot(ln_ref[...], w0_ref[...]) + b0_ref[...])
    h = _leaky(dot(h, w1_ref[...]) + b1_ref[...])
    h = _leaky(dot(h, w2_ref[...]) + b2_ref[...])
    o_ref[...] = dot(h, w3_ref[...]) + b3_ref[...]


def _enc_tables_body(nodes_ref, ts_ref, tr_ref, pe_ref, qe_ref):
    dot = lambda a, b: jnp.dot(a, b, preferred_element_type=jnp.float32)
    pe_ref[...] = dot(nodes_ref[...], ts_ref[...])
    qe_ref[...] = dot(nodes_ref[...], tr_ref[...])


def _full_call(body, args, out_shapes):
    full = lambda a: pl.BlockSpec(a.shape, lambda: (0,) * a.ndim)
    return pl.pallas_call(
        body,
        in_specs=[full(a) for a in args],
        out_specs=[pl.BlockSpec(s.shape, lambda: (0,) * len(s.shape)) for s in out_shapes],
        out_shape=out_shapes,
    )(*args)


def kernel(nodes, edges, senders, receivers, num_processing_steps, params):
    N = nodes.shape[0]
    E = senders.shape[0]
    N4 = N // PK
    EP = E // PK
    p = params
    zeros = jnp.zeros((N, LAT), jnp.float32)
    nodes_p = nodes.reshape(N4, nodes.shape[1] * PK)
    edges_p = edges.reshape(EP, edges.shape[1] * PK)
    npack = [jax.ShapeDtypeStruct((N4, LATP), jnp.float32)]
    tab = lambda t: t.reshape(N, LAT)
    epack = lambda g: g.reshape(EP, LATP)
    echunk = lambda x: x.reshape(E // _CHUNK, _CHUNK, LAT)
    ppack = lambda parts: parts.reshape(NC, N4, LATP)

    (We0, be0) = p['edge_enc_mlp'][0]
    T_e, T_s, T_r = We0[:16], We0[16:144], We0[144:272]
    Pe, Qe = _full_call(
        _enc_tables_body, [nodes_p, _blk(T_s), _blk(T_r)], npack * 2)
    ga0, gb0 = _sc_gather(tab(Pe), tab(Qe), senders, receivers)
    ee = _edge_mlp(edges_p, epack(ga0), epack(gb0), _tile(be0), T_e,
                   p['edge_enc_mlp'][1:], p['edge_enc_ln'])
    parts0 = ppack(_sc_scatter(echunk(ee), receivers, zeros))

    (Wn0, bn0) = p['node_enc_mlp'][0]
    (Wp0, bp0) = p['edge_proc_mlp'][0]
    W_sn, W_rn, W_le = Wp0[0:64], Wp0[64:128], Wp0[128:192]
    W_g0s, W_g0r, W_ee = Wp0[192:256], Wp0[256:320], Wp0[320:384]
    (Un0, bu0) = p['node_proc_mlp'][0]
    U_ln, U_agg, U_en, U_ga = Un0[0:64], Un0[64:128], Un0[128:192], Un0[192:256]
    (w1n, b1n), (w2n, b2n), (w3n, b3n) = p['node_enc_mlp'][1:]
    gn, betan = p['node_enc_ln']
    en, a0, b0, P2, Q2, c_n = _full_call(
        _enc_node_body,
        [nodes_p, parts0[0], parts0[1], _blk(Wn0[:128]), _blk(Wn0[128:]),
         _tile(bn0), _blk(w1n), _tile(b1n), _blk(w2n), _tile(b2n), _blk(w3n),
         _tile(b3n), _bgrp(), _tile(gn), _tile(betan),
         _blk(W_g0s), _blk(W_g0r), _blk(W_sn), _blk(W_rn), _blk(U_en),
         _blk(U_ga), _tile(bu0)],
        npack * 6)

    g2a, g2b = _sc_gather(tab(P2), tab(Q2), senders, receivers)
    c_e = _ce_pass(ee, epack(g2a), epack(g2b), W_ee, bp0)

    (w1, b1), (w2, b2), (w3, b3) = p['node_proc_mlp'][1:]
    gp, betap = p['node_proc_ln']
    step_consts = [_blk(U_ln), _blk(U_agg), _blk(w1), _tile(b1), _blk(w2),
                   _tile(b2), _blk(w3), _tile(b3), _bgrp(), _tile(gp),
                   _tile(betap), _blk(W_sn), _blk(W_rn)]

    def step(_, carry):
        ln, le, a, b = carry
        ga, gb = _sc_gather(tab(a), tab(b), senders, receivers)
        le2 = _edge_mlp(le, epack(ga), epack(gb), c_e, W_le,
                        p['edge_proc_mlp'][1:], p['edge_proc_ln'])
        parts = ppack(_sc_scatter(echunk(le2), receivers, zeros))
        ln2, a2, b2_ = _full_call(
            _step_node_body,
            [ln, parts[0], parts[1], c_n] + step_consts,
            npack * 3)
        return (ln2, le2, a2, b2_)

    ln, le, _, _ = lax.fori_loop(0, num_processing_steps, step, (en, ee, a0, b0))

    (Wd0, bd0), (wd1, bd1), (wd2, bd2), (wd3, bd3) = p['dec_mlp']
    D_OUT = wd3.shape[1]
    dec_p = _full_call(
        _dec_body,
        [ln, _blk(Wd0), _tile(bd0), _blk(wd1), _tile(bd1), _blk(wd2),
         _tile(bd2), _blk(wd3), _tile(bd3)],
        [jax.ShapeDtypeStruct((N4, D_OUT * PK), jnp.float32)])[0]
    return dec_p.reshape(N, D_OUT)


# speedup vs baseline: 1.2913x; 1.2913x over previous
import jax
import jax.numpy as jnp
from jax import lax
from jax.experimental import pallas as pl
from jax.experimental.pallas import tpu as pltpu
from jax.experimental.pallas import tpu_sc as plsc


LAT = 64
PK = 2
LATP = LAT * PK
NC = 2
NS = 16
NW = NC * NS


def _leaky(x):
    return jnp.where(x > 0, x, 0.01 * x)


def _blk(W):
    return jnp.kron(jnp.eye(PK, dtype=W.dtype), W)


def _tile(b):
    return jnp.tile(b, PK).reshape(1, b.shape[0] * PK)


def _bgrp():
    return jnp.kron(jnp.eye(PK, dtype=jnp.float32),
                    jnp.full((LAT, LAT), 1.0 / LAT, jnp.float32))


def _ln_packed(h, bgrp, g, beta):
    mu = jnp.dot(h, bgrp, preferred_element_type=jnp.float32)
    d = h - mu
    v = jnp.dot(d * d, bgrp, preferred_element_type=jnp.float32)
    return d * jax.lax.rsqrt(v + 1e-5) * g + beta


_CHUNK = 128
_GSUB = 4
_GRP = _CHUNK * _GSUB


def _sc_gather_body(tabA, tabB, senders, receivers, ga, gb,
                    sidx, ridx, bufA, bufB, semA, semB, semi):
    c = lax.axis_index("c")
    s = lax.axis_index("s")
    wid = c * NS + s
    E = senders.shape[0]
    ngrp = E // _GRP
    niter = (ngrp + NW - 1) // NW

    def group(j, _):
        g = wid + j * NW

        @pl.when(g < ngrp)
        def _():
            base = g * _GRP
            cpi1 = pltpu.async_copy(senders.at[pl.ds(base, _GRP)], sidx, semi)
            cpi2 = pltpu.async_copy(receivers.at[pl.ds(base, _GRP)], ridx, semi)
            cpi1.wait()
            cpi2.wait()
            cps = []
            for k in range(_GSUB):
                cps.append(pltpu.async_copy(
                    tabA.at[sidx.at[pl.ds(k * _CHUNK, _CHUNK)]],
                    bufA.at[k], semA))
                cps.append(pltpu.async_copy(
                    tabB.at[ridx.at[pl.ds(k * _CHUNK, _CHUNK)]],
                    bufB.at[k], semB))
            for cp in cps:
                cp.wait()
            cpo1 = pltpu.async_copy(bufA, ga.at[pl.ds(g * _GSUB, _GSUB)], semA)
            cpo2 = pltpu.async_copy(bufB, gb.at[pl.ds(g * _GSUB, _GSUB)], semB)
            cpo1.wait()
            cpo2.wait()
        return 0

    lax.fori_loop(0, niter, group, 0)


def _sc_gather(tabA, tabB, senders, receivers):
    E = senders.shape[0]
    mesh = plsc.VectorSubcoreMesh(core_axis_name="c", subcore_axis_name="s")
    out = jax.ShapeDtypeStruct((E // _CHUNK, _CHUNK, LAT), jnp.float32)
    return pl.kernel(
        _sc_gather_body,
        out_type=(out, out),
        mesh=mesh,
        scratch_types=[
            pltpu.VMEM((_GRP,), jnp.int32),
            pltpu.VMEM((_GRP,), jnp.int32),
            pltpu.VMEM((_GSUB, _CHUNK, LAT), jnp.float32),
            pltpu.VMEM((_GSUB, _CHUNK, LAT), jnp.float32),
            pltpu.SemaphoreType.DMA,
            pltpu.SemaphoreType.DMA,
            pltpu.SemaphoreType.DMA,
        ],
        compiler_params=pltpu.CompilerParams(use_tc_tiling_on_sc=False),
    )(tabA, tabB, senders, receivers)


def _sc_scatter_body(vals, receivers, zeros, out, ridx, vbuf, acc, sem):
    c = lax.axis_index("c")
    s = lax.axis_index("s")
    wid = c * NS + s
    nchunk = vals.shape[0]
    E = nchunk * _CHUNK
    N = zeros.shape[0]
    rows = N // NS
    niter = (nchunk + NW - 1) // NW

    pltpu.sync_copy(zeros.at[pl.ds(s * rows, rows)],
                    acc.at[pl.ds(s * rows, rows)])
    plsc.subcore_barrier()

    def chunk(j, _):
        ch = wid + j * NW

        @pl.when(ch < nchunk)
        def _():
            base = ch * _CHUNK
            cpi = pltpu.async_copy(receivers.at[pl.ds(base, _CHUNK)], ridx, sem)
            cpv = pltpu.async_copy(vals.at[ch], vbuf, sem)
            cpi.wait()
            cpv.wait()
            pltpu.sync_copy(vbuf, acc.at[ridx], add=True)
        return 0

    lax.fori_loop(0, niter, chunk, 0)
    plsc.subcore_barrier()
    pltpu.sync_copy(acc.at[pl.ds(s * rows, rows)], out.at[c, s])


def _sc_scatter(vals, receivers, zeros):
    N = zeros.shape[0]
    mesh = plsc.VectorSubcoreMesh(core_axis_name="c", subcore_axis_name="s")
    return pl.kernel(
        _sc_scatter_body,
        out_type=jax.ShapeDtypeStruct((NC, NS, N // NS, LAT), jnp.float32),
        mesh=mesh,
        scratch_types=[
            pltpu.VMEM((_CHUNK,), jnp.int32),
            pltpu.VMEM((_CHUNK, LAT), jnp.float32),
            pltpu.VMEM_SHARED((N, LAT), jnp.float32),
            pltpu.SemaphoreType.DMA,
        ],
        compiler_params=pltpu.CompilerParams(use_tc_tiling_on_sc=False),
    )(vals, receivers, zeros)


def _edge_mlp_body(x1_ref, ga_ref, gb_ref, c_ref, m_ref, w1_ref, b1_ref,
                   w2_ref, b2_ref, w3_ref, b3_ref, bgrp_ref, g_ref, beta_ref,
                   o_ref):
    h = jnp.dot(x1_ref[...], m_ref[...], preferred_element_type=jnp.float32)
    h = h + ga_ref[...] + gb_ref[...] + c_ref[...]
    h = _leaky(h)
    h = _leaky(jnp.dot(h, w1_ref[...], preferred_element_type=jnp.float32) + b1_ref[...])
    h = _leaky(jnp.dot(h, w2_ref[...], preferred_element_type=jnp.float32) + b2_ref[...])
    h = jnp.dot(h, w3_ref[...], preferred_element_type=jnp.float32) + b3_ref[...]
    o_ref[...] = _ln_packed(h, bgrp_ref[...], g_ref[...], beta_ref[...])


def _edge_mlp(x1, ga, gb, c, M, tail_params, ln, block_rows=1000):
    EP, KP = x1.shape
    grid = (EP // block_rows,)
    (w1, b1), (w2, b2), (w3, b3) = tail_params
    g, beta = ln
    row_spec = pl.BlockSpec((block_rows, LATP), lambda i: (i, 0))
    x1_spec = pl.BlockSpec((block_rows, KP), lambda i: (i, 0))
    c_spec = (row_spec if c.shape[0] == EP
              else pl.BlockSpec((1, LATP), lambda i: (0, 0)))
    full = lambda a: pl.BlockSpec(a.shape, lambda i: (0,) * a.ndim)
    small = [_blk(M), _blk(w1), _tile(b1), _blk(w2), _tile(b2),
             _blk(w3), _tile(b3), _bgrp(), _tile(g), _tile(beta)]
    return pl.pallas_call(
        _edge_mlp_body,
        grid=grid,
        in_specs=[x1_spec, row_spec, row_spec, c_spec] + [full(a) for a in small],
        out_specs=row_spec,
        out_shape=jax.ShapeDtypeStruct((EP, LATP), jnp.float32),
    )(x1, ga, gb, c, *small)


def _ce_body(ee_ref, g2a_ref, g2b_ref, wee_ref, bp0_ref, o_ref):
    o_ref[...] = (jnp.dot(ee_ref[...], wee_ref[...],
                          preferred_element_type=jnp.float32)
                  + g2a_ref[...] + g2b_ref[...] + bp0_ref[...])


def _ce_pass(ee, g2a, g2b, W_ee, bp0, block_rows=1000):
    EP = ee.shape[0]
    row_spec = pl.BlockSpec((block_rows, LATP), lambda i: (i, 0))
    return pl.pallas_call(
        _ce_body,
        grid=(EP // block_rows,),
        in_specs=[row_spec, row_spec, row_spec,
                  pl.BlockSpec((LATP, LATP), lambda i: (0, 0)),
                  pl.BlockSpec((1, LATP), lambda i: (0, 0))],
        out_specs=row_spec,
        out_shape=jax.ShapeDtypeStruct((EP, LATP), jnp.float32),
    )(ee, g2a, g2b, _blk(W_ee), _tile(bp0))


def _enc_node_body(nodes_ref, p0_ref, p1_ref, vn_ref, va_ref, b0_ref, w1_ref,
                   b1_ref, w2_ref, b2_ref, w3_ref, b3_ref, bgrp_ref, g_ref,
                   beta_ref, wg0s_ref, wg0r_ref, wsn_ref, wrn_ref, uen_ref,
                   uga_ref, bu0_ref,
                   en_ref, a0_ref, b0out_ref, p2_ref, q2_ref, cn_ref):
    dot = lambda a, b: jnp.dot(a, b, preferred_element_type=jnp.float32)
    agg0 = p0_ref[...] + p1_ref[...]
    h = dot(nodes_ref[...], vn_ref[...]) + dot(agg0, va_ref[...]) + b0_ref[...]
    h = _leaky(h)
    h = _leaky(dot(h, w1_ref[...]) + b1_ref[...])
    h = _leaky(dot(h, w2_ref[...]) + b2_ref[...])
    h = dot(h, w3_ref[...]) + b3_ref[...]
    en = _ln_packed(h, bgrp_ref[...], g_ref[...], beta_ref[...])
    en_ref[...] = en
    p2_ref[...] = dot(en, wg0s_ref[...])
    q2_ref[...] = dot(en, wg0r_ref[...])
    a0_ref[...] = dot(en, wsn_ref[...])
    b0out_ref[...] = dot(en, wrn_ref[...])
    cn_ref[...] = dot(en, uen_ref[...]) + dot(agg0, uga_ref[...]) + bu0_ref[...]


def _step_node_body(ln_ref, p0_ref, p1_ref, cn_ref, uln_ref, uagg_ref,
                    w1_ref, b1_ref, w2_ref, b2_ref, w3_ref, b3_ref,
                    bgrp_ref, g_ref, beta_ref, wsn_ref, wrn_ref,
                    lnout_ref, aout_ref, bout_ref):
    dot = lambda a, b: jnp.dot(a, b, preferred_element_type=jnp.float32)
    agg = p0_ref[...] + p1_ref[...]
    h = dot(ln_ref[...], uln_ref[...]) + dot(agg, uagg_ref[...]) + cn_ref[...]
    h = _leaky(h)
    h = _leaky(dot(h, w1_ref[...]) + b1_ref[...])
    h = _leaky(dot(h, w2_ref[...]) + b2_ref[...])
    h = dot(h, w3_ref[...]) + b3_ref[...]
    ln2 = _ln_packed(h, bgrp_ref[...], g_ref[...], beta_ref[...])
    lnout_ref[...] = ln2
    aout_ref[...] = dot(ln2, wsn_ref[...])
    bout_ref[...] = dot(ln2, wrn_ref[...])


def _dec_body(ln_ref, w0_ref, b0_ref, w1_ref, b1_ref, w2_ref, b2_ref,
              w3_ref, b3_ref, o_ref):
    dot = lambda a, b: jnp.dot(a, b, preferred_element_type=jnp.float32)
    h = _leaky(dot(ln_ref[...], w0_ref[...]) + b0_ref[...])
    h = _leaky(dot(h, w1_ref[...]) + b1_ref[...])
    h = _leaky(dot(h, w2_ref[...]) + b2_ref[...])
    o_ref[...] = dot(h, w3_ref[...]) + b3_ref[...]


def _enc_tables_body(nodes_ref, ts_ref, tr_ref, pe_ref, qe_ref):
    dot = lambda a, b: jnp.dot(a, b, preferred_element_type=jnp.float32)
    pe_ref[...] = dot(nodes_ref[...], ts_ref[...])
    qe_ref[...] = dot(nodes_ref[...], tr_ref[...])


def _full_call(body, args, out_shapes):
    full = lambda a: pl.BlockSpec(a.shape, lambda: (0,) * a.ndim)
    return pl.pallas_call(
        body,
        in_specs=[full(a) for a in args],
        out_specs=[pl.BlockSpec(s.shape, lambda: (0,) * len(s.shape)) for s in out_shapes],
        out_shape=out_shapes,
    )(*args)


def kernel(nodes, edges, senders, receivers, num_processing_steps, params):
    N = nodes.shape[0]
    E = senders.shape[0]
    N4 = N // PK
    EP = E // PK
    p = params
    zeros = jnp.zeros((N, LAT), jnp.float32)
    nodes_p = nodes.reshape(N4, nodes.shape[1] * PK)
    edges_p = edges.reshape(EP, edges.shape[1] * PK)
    npack = [jax.ShapeDtypeStruct((N4, LATP), jnp.float32)]
    tab = lambda t: t.reshape(N, LAT)
    epack = lambda g: g.reshape(EP, LATP)
    echunk = lambda x: x.reshape(E // _CHUNK, _CHUNK, LAT)
    ppack = lambda parts: parts.reshape(NC, N4, LATP)

    (We0, be0) = p['edge_enc_mlp'][0]
    T_e, T_s, T_r = We0[:16], We0[16:144], We0[144:272]
    Pe, Qe = _full_call(
        _enc_tables_body, [nodes_p, _blk(T_s), _blk(T_r)], npack * 2)
    ga0, gb0 = _sc_gather(tab(Pe), tab(Qe), senders, receivers)
    ee = _edge_mlp(edges_p, epack(ga0), epack(gb0), _tile(be0), T_e,
                   p['edge_enc_mlp'][1:], p['edge_enc_ln'])
    parts0 = ppack(_sc_scatter(echunk(ee), receivers, zeros))

    (Wn0, bn0) = p['node_enc_mlp'][0]
    (Wp0, bp0) = p['edge_proc_mlp'][0]
    W_sn, W_rn, W_le = Wp0[0:64], Wp0[64:128], Wp0[128:192]
    W_g0s, W_g0r, W_ee = Wp0[192:256], Wp0[256:320], Wp0[320:384]
    (Un0, bu0) = p['node_proc_mlp'][0]
    U_ln, U_agg, U_en, U_ga = Un0[0:64], Un0[64:128], Un0[128:192], Un0[192:256]
    (w1n, b1n), (w2n, b2n), (w3n, b3n) = p['node_enc_mlp'][1:]
    gn, betan = p['node_enc_ln']
    en, a0, b0, P2, Q2, c_n = _full_call(
        _enc_node_body,
        [nodes_p, parts0[0], parts0[1], _blk(Wn0[:128]), _blk(Wn0[128:]),
         _tile(bn0), _blk(w1n), _tile(b1n), _blk(w2n), _tile(b2n), _blk(w3n),
         _tile(b3n), _bgrp(), _tile(gn), _tile(betan),
         _blk(W_g0s), _blk(W_g0r), _blk(W_sn), _blk(W_rn), _blk(U_en),
         _blk(U_ga), _tile(bu0)],
        npack * 6)

    g2a, g2b = _sc_gather(tab(P2), tab(Q2), senders, receivers)
    c_e = _ce_pass(ee, epack(g2a), epack(g2b), W_ee, bp0)

    (w1, b1), (w2, b2), (w3, b3) = p['node_proc_mlp'][1:]
    gp, betap = p['node_proc_ln']
    step_consts = [_blk(U_ln), _blk(U_agg), _blk(w1), _tile(b1), _blk(w2),
                   _tile(b2), _blk(w3), _tile(b3), _bgrp(), _tile(gp),
                   _tile(betap), _blk(W_sn), _blk(W_rn)]

    def step(_, carry):
        ln, le, a, b = carry
        ga, gb = _sc_gather(tab(a), tab(b), senders, receivers)
        le2 = _edge_mlp(le, epack(ga), epack(gb), c_e, W_le,
                        p['edge_proc_mlp'][1:], p['edge_proc_ln'])
        parts = ppack(_sc_scatter(echunk(le2), receivers, zeros))
        ln2, a2, b2_ = _full_call(
            _step_node_body,
            [ln, parts[0], parts[1], c_n] + step_consts,
            npack * 3)
        return (ln2, le2, a2, b2_)

    ln, le, _, _ = lax.fori_loop(0, num_processing_steps, step, (en, ee, a0, b0))

    (Wd0, bd0), (wd1, bd1), (wd2, bd2), (wd3, bd3) = p['dec_mlp']
    D_OUT = wd3.shape[1]
    dec_p = _full_call(
        _dec_body,
        [ln, _blk(Wd0), _tile(bd0), _blk(wd1), _tile(bd1), _blk(wd2),
         _tile(bd2), _blk(wd3), _tile(bd3)],
        [jax.ShapeDtypeStruct((N4, D_OUT * PK), jnp.float32)])[0]
    return dec_p.reshape(N, D_OUT)


# speedup vs baseline: 1.3455x; 1.0419x over previous
import jax
import jax.numpy as jnp
from jax import lax
from jax.experimental import pallas as pl
from jax.experimental.pallas import tpu as pltpu
from jax.experimental.pallas import tpu_sc as plsc


LAT = 64
PK = 2
LATP = LAT * PK
NC = 2
NS = 16
NW = NC * NS


def _leaky(x):
    return jnp.where(x > 0, x, 0.01 * x)


def _blk(W):
    return jnp.kron(jnp.eye(PK, dtype=W.dtype), W)


def _tile(b):
    return jnp.tile(b, PK).reshape(1, b.shape[0] * PK)


def _bgrp():
    return jnp.kron(jnp.eye(PK, dtype=jnp.float32),
                    jnp.full((LAT, LAT), 1.0 / LAT, jnp.float32))


def _ln_packed(h, bgrp, g, beta):
    mu = jnp.dot(h, bgrp, preferred_element_type=jnp.float32)
    d = h - mu
    v = jnp.dot(d * d, bgrp, preferred_element_type=jnp.float32)
    return d * jax.lax.rsqrt(v + 1e-5) * g + beta


_CHUNK = 128
_GSUB = 4
_GRP = _CHUNK * _GSUB


def _sc_gather_body(tabA, tabB, senders, receivers, ga, gb,
                    sidx, ridx, bufA, bufB, tAs, semA, semB, semi):
    c = lax.axis_index("c")
    s = lax.axis_index("s")
    wid = c * NS + s
    E = senders.shape[0]
    N = tabA.shape[0]
    trows = N // NS
    ngrp = E // _GRP
    niter = (ngrp + NW - 1) // NW

    pltpu.async_copy(tabA.at[pl.ds(s * trows, trows)],
                     tAs.at[pl.ds(s * trows, trows)], semA).wait()
    plsc.subcore_barrier()

    def group(j, _):
        g = wid + j * NW

        @pl.when(g < ngrp)
        def _():
            base = g * _GRP
            cpi1 = pltpu.async_copy(senders.at[pl.ds(base, _GRP)], sidx, semi)
            cpi2 = pltpu.async_copy(receivers.at[pl.ds(base, _GRP)], ridx, semi)
            cpi1.wait()
            cpi2.wait()
            cps = []
            for k in range(_GSUB):
                cps.append(pltpu.async_copy(
                    tAs.at[sidx.at[pl.ds(k * _CHUNK, _CHUNK)]],
                    bufA.at[k], semA))
                cps.append(pltpu.async_copy(
                    tabB.at[ridx.at[pl.ds(k * _CHUNK, _CHUNK)]],
                    bufB.at[k], semB))
            for cp in cps:
                cp.wait()
            cpo1 = pltpu.async_copy(bufA, ga.at[pl.ds(g * _GSUB, _GSUB)], semA)
            cpo2 = pltpu.async_copy(bufB, gb.at[pl.ds(g * _GSUB, _GSUB)], semB)
            cpo1.wait()
            cpo2.wait()
        return 0

    lax.fori_loop(0, niter, group, 0)


def _sc_gather(tabA, tabB, senders, receivers):
    E = senders.shape[0]
    N = tabA.shape[0]
    mesh = plsc.VectorSubcoreMesh(core_axis_name="c", subcore_axis_name="s")
    out = jax.ShapeDtypeStruct((E // _CHUNK, _CHUNK, LAT), jnp.float32)
    return pl.kernel(
        _sc_gather_body,
        out_type=(out, out),
        mesh=mesh,
        scratch_types=[
            pltpu.VMEM((_GRP,), jnp.int32),
            pltpu.VMEM((_GRP,), jnp.int32),
            pltpu.VMEM((_GSUB, _CHUNK, LAT), jnp.float32),
            pltpu.VMEM((_GSUB, _CHUNK, LAT), jnp.float32),
            pltpu.VMEM_SHARED((N, LAT), jnp.float32),
            pltpu.SemaphoreType.DMA,
            pltpu.SemaphoreType.DMA,
            pltpu.SemaphoreType.DMA,
        ],
        compiler_params=pltpu.CompilerParams(use_tc_tiling_on_sc=False),
    )(tabA, tabB, senders, receivers)


def _sc_scatter_body(vals, receivers, zeros, out, ridx, vbuf, acc, sem):
    c = lax.axis_index("c")
    s = lax.axis_index("s")
    wid = c * NS + s
    nchunk = vals.shape[0]
    E = nchunk * _CHUNK
    N = zeros.shape[0]
    rows = N // NS
    niter = (nchunk + NW - 1) // NW

    pltpu.sync_copy(zeros.at[pl.ds(s * rows, rows)],
                    acc.at[pl.ds(s * rows, rows)])
    plsc.subcore_barrier()

    def chunk(j, _):
        ch = wid + j * NW

        @pl.when(ch < nchunk)
        def _():
            base = ch * _CHUNK
            cpi = pltpu.async_copy(receivers.at[pl.ds(base, _CHUNK)], ridx, sem)
            cpv = pltpu.async_copy(vals.at[ch], vbuf, sem)
            cpi.wait()
            cpv.wait()
            pltpu.sync_copy(vbuf, acc.at[ridx], add=True)
        return 0

    lax.fori_loop(0, niter, chunk, 0)
    plsc.subcore_barrier()
    pltpu.sync_copy(acc.at[pl.ds(s * rows, rows)], out.at[c, s])


def _sc_scatter(vals, receivers, zeros):
    N = zeros.shape[0]
    mesh = plsc.VectorSubcoreMesh(core_axis_name="c", subcore_axis_name="s")
    return pl.kernel(
        _sc_scatter_body,
        out_type=jax.ShapeDtypeStruct((NC, NS, N // NS, LAT), jnp.float32),
        mesh=mesh,
        scratch_types=[
            pltpu.VMEM((_CHUNK,), jnp.int32),
            pltpu.VMEM((_CHUNK, LAT), jnp.float32),
            pltpu.VMEM_SHARED((N, LAT), jnp.float32),
            pltpu.SemaphoreType.DMA,
        ],
        compiler_params=pltpu.CompilerParams(use_tc_tiling_on_sc=False),
    )(vals, receivers, zeros)


def _edge_mlp_body(x1_ref, ga_ref, gb_ref, c_ref, m_ref, w1_ref, b1_ref,
                   w2_ref, b2_ref, w3_ref, b3_ref, bgrp_ref, g_ref, beta_ref,
                   o_ref):
    h = jnp.dot(x1_ref[...], m_ref[...], preferred_element_type=jnp.float32)
    h = h + ga_ref[...] + gb_ref[...] + c_ref[...]
    h = _leaky(h)
    h = _leaky(jnp.dot(h, w1_ref[...], preferred_element_type=jnp.float32) + b1_ref[...])
    h = _leaky(jnp.dot(h, w2_ref[...], preferred_element_type=jnp.float32) + b2_ref[...])
    h = jnp.dot(h, w3_ref[...], preferred_element_type=jnp.float32) + b3_ref[...]
    o_ref[...] = _ln_packed(h, bgrp_ref[...], g_ref[...], beta_ref[...])


def _edge_mlp(x1, ga, gb, c, M, tail_params, ln, block_rows=1000):
    EP, KP = x1.shape
    grid = (EP // block_rows,)
    (w1, b1), (w2, b2), (w3, b3) = tail_params
    g, beta = ln
    row_spec = pl.BlockSpec((block_rows, LATP), lambda i: (i, 0))
    x1_spec = pl.BlockSpec((block_rows, KP), lambda i: (i, 0))
    c_spec = (row_spec if c.shape[0] == EP
              else pl.BlockSpec((1, LATP), lambda i: (0, 0)))
    full = lambda a: pl.BlockSpec(a.shape, lambda i: (0,) * a.ndim)
    small = [_blk(M), _blk(w1), _tile(b1), _blk(w2), _tile(b2),
             _blk(w3), _tile(b3), _bgrp(), _tile(g), _tile(beta)]
    return pl.pallas_call(
        _edge_mlp_body,
        grid=grid,
        in_specs=[x1_spec, row_spec, row_spec, c_spec] + [full(a) for a in small],
        out_specs=row_spec,
        out_shape=jax.ShapeDtypeStruct((EP, LATP), jnp.float32),
    )(x1, ga, gb, c, *small)


def _ce_body(ee_ref, g2a_ref, g2b_ref, wee_ref, bp0_ref, o_ref):
    o_ref[...] = (jnp.dot(ee_ref[...], wee_ref[...],
                          preferred_element_type=jnp.float32)
                  + g2a_ref[...] + g2b_ref[...] + bp0_ref[...])


def _ce_pass(ee, g2a, g2b, W_ee, bp0, block_rows=1000):
    EP = ee.shape[0]
    row_spec = pl.BlockSpec((block_rows, LATP), lambda i: (i, 0))
    return pl.pallas_call(
        _ce_body,
        grid=(EP // block_rows,),
        in_specs=[row_spec, row_spec, row_spec,
                  pl.BlockSpec((LATP, LATP), lambda i: (0, 0)),
                  pl.BlockSpec((1, LATP), lambda i: (0, 0))],
        out_specs=row_spec,
        out_shape=jax.ShapeDtypeStruct((EP, LATP), jnp.float32),
    )(ee, g2a, g2b, _blk(W_ee), _tile(bp0))


def _enc_node_body(nodes_ref, p0_ref, p1_ref, vn_ref, va_ref, b0_ref, w1_ref,
                   b1_ref, w2_ref, b2_ref, w3_ref, b3_ref, bgrp_ref, g_ref,
                   beta_ref, wg0s_ref, wg0r_ref, wsn_ref, wrn_ref, uen_ref,
                   uga_ref, bu0_ref,
                   en_ref, a0_ref, b0out_ref, p2_ref, q2_ref, cn_ref):
    dot = lambda a, b: jnp.dot(a, b, preferred_element_type=jnp.float32)
    agg0 = p0_ref[...] + p1_ref[...]
    h = dot(nodes_ref[...], vn_ref[...]) + dot(agg0, va_ref[...]) + b0_ref[...]
    h = _leaky(h)
    h = _leaky(dot(h, w1_ref[...]) + b1_ref[...])
    h = _leaky(dot(h, w2_ref[...]) + b2_ref[...])
    h = dot(h, w3_ref[...]) + b3_ref[...]
    en = _ln_packed(h, bgrp_ref[...], g_ref[...], beta_ref[...])
    en_ref[...] = en
    p2_ref[...] = dot(en, wg0s_ref[...])
    q2_ref[...] = dot(en, wg0r_ref[...])
    a0_ref[...] = dot(en, wsn_ref[...])
    b0out_ref[...] = dot(en, wrn_ref[...])
    cn_ref[...] = dot(en, uen_ref[...]) + dot(agg0, uga_ref[...]) + bu0_ref[...]


def _step_node_body(ln_ref, p0_ref, p1_ref, cn_ref, uln_ref, uagg_ref,
                    w1_ref, b1_ref, w2_ref, b2_ref, w3_ref, b3_ref,
                    bgrp_ref, g_ref, beta_ref, wsn_ref, wrn_ref,
                    lnout_ref, aout_ref, bout_ref):
    dot = lambda a, b: jnp.dot(a, b, preferred_element_type=jnp.float32)
    agg = p0_ref[...] + p1_ref[...]
    h = dot(ln_ref[...], uln_ref[...]) + dot(agg, uagg_ref[...]) + cn_ref[...]
    h = _leaky(h)
    h = _leaky(dot(h, w1_ref[...]) + b1_ref[...])
    h = _leaky(dot(h, w2_ref[...]) + b2_ref[...])
    h = dot(h, w3_ref[...]) + b3_ref[...]
    ln2 = _ln_packed(h, bgrp_ref[...], g_ref[...], beta_ref[...])
    lnout_ref[...] = ln2
    aout_ref[...] = dot(ln2, wsn_ref[...])
    bout_ref[...] = dot(ln2, wrn_ref[...])


def _dec_body(ln_ref, w0_ref, b0_ref, w1_ref, b1_ref, w2_ref, b2_ref,
              w3_ref, b3_ref, o_ref):
    dot = lambda a, b: jnp.dot(a, b, preferred_element_type=jnp.float32)
    h = _leaky(dot(ln_ref[...], w0_ref[...]) + b0_ref[...])
    h = _leaky(dot(h, w1_ref[...]) + b1_ref[...])
    h = _leaky(dot(h, w2_ref[...]) + b2_ref[...])
    o_ref[...] = dot(h, w3_ref[...]) + b3_ref[...]


def _enc_tables_body(nodes_ref, ts_ref, tr_ref, pe_ref, qe_ref):
    dot = lambda a, b: jnp.dot(a, b, preferred_element_type=jnp.float32)
    pe_ref[...] = dot(nodes_ref[...], ts_ref[...])
    qe_ref[...] = dot(nodes_ref[...], tr_ref[...])


def _full_call(body, args, out_shapes):
    full = lambda a: pl.BlockSpec(a.shape, lambda: (0,) * a.ndim)
    return pl.pallas_call(
        body,
        in_specs=[full(a) for a in args],
        out_specs=[pl.BlockSpec(s.shape, lambda: (0,) * len(s.shape)) for s in out_shapes],
        out_shape=out_shapes,
    )(*args)


def kernel(nodes, edges, senders, receivers, num_processing_steps, params):
    N = nodes.shape[0]
    E = senders.shape[0]
    N4 = N // PK
    EP = E // PK
    p = params
    zeros = jnp.zeros((N, LAT), jnp.float32)
    nodes_p = nodes.reshape(N4, nodes.shape[1] * PK)
    edges_p = edges.reshape(EP, edges.shape[1] * PK)
    npack = [jax.ShapeDtypeStruct((N4, LATP), jnp.float32)]
    tab = lambda t: t.reshape(N, LAT)
    epack = lambda g: g.reshape(EP, LATP)
    echunk = lambda x: x.reshape(E // _CHUNK, _CHUNK, LAT)
    ppack = lambda parts: parts.reshape(NC, N4, LATP)

    (We0, be0) = p['edge_enc_mlp'][0]
    T_e, T_s, T_r = We0[:16], We0[16:144], We0[144:272]
    Pe, Qe = _full_call(
        _enc_tables_body, [nodes_p, _blk(T_s), _blk(T_r)], npack * 2)
    ga0, gb0 = _sc_gather(tab(Pe), tab(Qe), senders, receivers)
    ee = _edge_mlp(edges_p, epack(ga0), epack(gb0), _tile(be0), T_e,
                   p['edge_enc_mlp'][1:], p['edge_enc_ln'])
    parts0 = ppack(_sc_scatter(echunk(ee), receivers, zeros))

    (Wn0, bn0) = p['node_enc_mlp'][0]
    (Wp0, bp0) = p['edge_proc_mlp'][0]
    W_sn, W_rn, W_le = Wp0[0:64], Wp0[64:128], Wp0[128:192]
    W_g0s, W_g0r, W_ee = Wp0[192:256], Wp0[256:320], Wp0[320:384]
    (Un0, bu0) = p['node_proc_mlp'][0]
    U_ln, U_agg, U_en, U_ga = Un0[0:64], Un0[64:128], Un0[128:192], Un0[192:256]
    (w1n, b1n), (w2n, b2n), (w3n, b3n) = p['node_enc_mlp'][1:]
    gn, betan = p['node_enc_ln']
    en, a0, b0, P2, Q2, c_n = _full_call(
        _enc_node_body,
        [nodes_p, parts0[0], parts0[1], _blk(Wn0[:128]), _blk(Wn0[128:]),
         _tile(bn0), _blk(w1n), _tile(b1n), _blk(w2n), _tile(b2n), _blk(w3n),
         _tile(b3n), _bgrp(), _tile(gn), _tile(betan),
         _blk(W_g0s), _blk(W_g0r), _blk(W_sn), _blk(W_rn), _blk(U_en),
         _blk(U_ga), _tile(bu0)],
        npack * 6)

    g2a, g2b = _sc_gather(tab(P2), tab(Q2), senders, receivers)
    c_e = _ce_pass(ee, epack(g2a), epack(g2b), W_ee, bp0)

    (w1, b1), (w2, b2), (w3, b3) = p['node_proc_mlp'][1:]
    gp, betap = p['node_proc_ln']
    step_consts = [_blk(U_ln), _blk(U_agg), _blk(w1), _tile(b1), _blk(w2),
                   _tile(b2), _blk(w3), _tile(b3), _bgrp(), _tile(gp),
                   _tile(betap), _blk(W_sn), _blk(W_rn)]

    def step(_, carry):
        ln, le, a, b = carry
        ga, gb = _sc_gather(tab(a), tab(b), senders, receivers)
        le2 = _edge_mlp(le, epack(ga), epack(gb), c_e, W_le,
                        p['edge_proc_mlp'][1:], p['edge_proc_ln'])
        parts = ppack(_sc_scatter(echunk(le2), receivers, zeros))
        ln2, a2, b2_ = _full_call(
            _step_node_body,
            [ln, parts[0], parts[1], c_n] + step_consts,
            npack * 3)
        return (ln2, le2, a2, b2_)

    ln, le, _, _ = lax.fori_loop(0, num_processing_steps, step, (en, ee, a0, b0))

    (Wd0, bd0), (wd1, bd1), (wd2, bd2), (wd3, bd3) = p['dec_mlp']
    D_OUT = wd3.shape[1]
    dec_p = _full_call(
        _dec_body,
        [ln, _blk(Wd0), _tile(bd0), _blk(wd1), _tile(bd1), _blk(wd2),
         _tile(bd2), _blk(wd3), _tile(bd3)],
        [jax.ShapeDtypeStruct((N4, D_OUT * PK), jnp.float32)])[0]
    return dec_p.reshape(N, D_OUT)
